# baseline (speedup 1.0000x reference)
import functools

import jax
import jax.numpy as jnp
from jax import lax
from jax.experimental import pallas as pl
from jax.experimental.pallas import tpu as pltpu
from jax.experimental.pallas import tpu_sc as plsc

NUM_EXPERTS = 8
TOP_K = 2
SEQ = 2048
D_MODEL = 1024
MLP_DIM = 4096

BT = 256
FB = 512
NF = MLP_DIM // FB
NB = (SEQ * TOP_K) // BT + NUM_EXPERTS
R = NB * BT


_NW = 32
_ROW_I32 = D_MODEL // 2


def _sc_gather_rows(table_i32, idx, n_rows):
    b_per_w = n_rows // _NW
    ch = 64 if b_per_w % 64 == 0 else b_per_w
    nch = b_per_w // ch
    mesh = plsc.VectorSubcoreMesh(core_axis_name="c", subcore_axis_name="s")

    @functools.partial(
        pl.kernel,
        mesh=mesh,
        out_type=jax.ShapeDtypeStruct((n_rows, _ROW_I32), jnp.int32),
        scratch_types=[
            pltpu.VMEM((b_per_w,), jnp.int32),
            pltpu.VMEM((2, ch, _ROW_I32), jnp.int32),
            pltpu.SemaphoreType.DMA,
            pltpu.SemaphoreType.DMA,
        ],
    )
    def k(table_hbm, idx_hbm, out_hbm, idx_v, rows_v, sem0, sem1):
        wid = lax.axis_index("s") * 2 + lax.axis_index("c")
        base = wid * b_per_w
        pltpu.sync_copy(idx_hbm.at[pl.ds(base, b_per_w)], idx_v)
        sems = (sem0, sem1)
        copies = [None, None]
        for c in range(nch):
            s = c % 2
            copies[s] = pltpu.async_copy(
                table_hbm.at[idx_v.at[pl.ds(c * ch, ch)]],
                rows_v.at[s], sems[s])
            if c > 0:
                copies[1 - s].wait()
                pltpu.sync_copy(rows_v.at[1 - s],
                                out_hbm.at[pl.ds(base + (c - 1) * ch, ch)])
        copies[(nch - 1) % 2].wait()
        pltpu.sync_copy(rows_v.at[(nch - 1) % 2],
                        out_hbm.at[pl.ds(base + (nch - 1) * ch, ch)])

    return k(table_i32, idx)


def _bf16_as_i32(a2d):
    n, d = a2d.shape
    return jax.lax.bitcast_convert_type(
        a2d.reshape(n, d // 2, 2), jnp.int32)


def _i32_as_bf16(a2d):
    n, d = a2d.shape
    return jax.lax.bitcast_convert_type(a2d, jnp.bfloat16).reshape(n, 2 * d)


def _moe_mlp_kernel(s_ref, x_ref, w0_ref, w1_ref, wo_ref, o_ref, acc_ref):
    j = pl.program_id(0)
    i = pl.program_id(1)
    nb = s_ref[NB]

    @pl.when(i < nb)
    def _():
        x = x_ref[pl.ds(i * BT, BT), :]
        h0 = jnp.dot(x, w0_ref[0], preferred_element_type=jnp.float32)
        h1 = jnp.dot(x, w1_ref[0], preferred_element_type=jnp.float32)
        h = jax.nn.silu(h0) * h1
        y = jnp.dot(h, wo_ref[0], preferred_element_type=jnp.float32)

        @pl.when(j == 0)
        def _():
            acc_ref[pl.ds(i * BT, BT), :] = y

        @pl.when(j > 0)
        def _():
            acc_ref[pl.ds(i * BT, BT), :] += y

        @pl.when(j == NF - 1)
        def _():
            o_ref[...] = acc_ref[pl.ds(i * BT, BT), :].astype(jnp.bfloat16)


def _grouped_mlp(meta, x_g, wi_0, wi_1, wo):
    grid_spec = pltpu.PrefetchScalarGridSpec(
        num_scalar_prefetch=1,
        grid=(NF, NB),
        in_specs=[
            pl.BlockSpec((R, D_MODEL), lambda j, i, s: (0, 0)),
            pl.BlockSpec((1, D_MODEL, FB), lambda j, i, s: (s[i], 0, j)),
            pl.BlockSpec((1, D_MODEL, FB), lambda j, i, s: (s[i], 0, j)),
            pl.BlockSpec((1, FB, D_MODEL), lambda j, i, s: (s[i], j, 0)),
        ],
        out_specs=pl.BlockSpec(
            (BT, D_MODEL),
            lambda j, i, s: (jnp.where(j == NF - 1, i, 0), 0)),
        scratch_shapes=[pltpu.VMEM((R, D_MODEL), jnp.float32)],
    )
    return pl.pallas_call(
        _moe_mlp_kernel,
        grid_spec=grid_spec,
        out_shape=jax.ShapeDtypeStruct((R, D_MODEL), jnp.bfloat16),
        compiler_params=pltpu.CompilerParams(
            dimension_semantics=("arbitrary", "arbitrary"),
        ),
    )(meta, x_g, wi_0, wi_1, wo)


def kernel(inputs, gate_w, wi_0, wi_1, wo):
    x = inputs.reshape(SEQ, D_MODEL)

    logits = x @ gate_w
    e0 = jnp.argmax(logits, axis=-1).astype(jnp.int32)
    v0 = jnp.max(logits, axis=-1)
    masked = jnp.where(
        jax.nn.one_hot(e0, NUM_EXPERTS, dtype=jnp.bool_), -jnp.inf, logits)
    e1 = jnp.argmax(masked, axis=-1).astype(jnp.int32)
    v1 = jnp.max(masked, axis=-1)
    p1 = jax.nn.sigmoid(v1 - v0)
    top_w = jnp.stack([1.0 - p1, p1], axis=-1)
    experts_flat = jnp.stack([e0, e1], axis=-1).reshape(-1)

    onehot = (experts_flat[:, None] ==
              jnp.arange(NUM_EXPERTS)[None, :]).astype(jnp.int32)
    csum = jnp.cumsum(onehot, axis=0)
    counts = csum[-1]
    ranks = jnp.take_along_axis(csum, experts_flat[:, None], axis=1)[:, 0] - 1
    padded_counts = ((counts + BT - 1) // BT) * BT
    padded_offsets = jnp.concatenate(
        [jnp.zeros((1,), jnp.int32), jnp.cumsum(padded_counts)[:-1]]
    ).astype(jnp.int32)
    pos = padded_offsets[experts_flat] + ranks
    num_blocks = (padded_offsets[-1] + padded_counts[-1]) // BT

    token_of = jnp.arange(SEQ * TOP_K, dtype=jnp.int32) // TOP_K
    gather_idx = jnp.zeros((R,), jnp.int32).at[pos].set(
        token_of, unique_indices=True, mode="promise_in_bounds")
    block_expert = (
        jnp.searchsorted(padded_offsets,
                         jnp.arange(NB, dtype=jnp.int32) * BT, side="right")
        - 1
    ).astype(jnp.int32)
    meta = jnp.concatenate(
        [block_expert, num_blocks.reshape(1).astype(jnp.int32)])

    x_i32 = _bf16_as_i32(x.astype(jnp.bfloat16))
    x_g = _i32_as_bf16(_sc_gather_rows(x_i32, gather_idx, R))

    y_g = _grouped_mlp(meta, x_g, wi_0, wi_1, wo)

    y_pairs = _i32_as_bf16(
        _sc_gather_rows(_bf16_as_i32(y_g), pos, SEQ * TOP_K))
    out = (top_w[:, :, None] *
           y_pairs.reshape(SEQ, TOP_K, D_MODEL)).sum(axis=1)
    return out.reshape(1, SEQ, D_MODEL)


# speedup vs baseline: 4.2691x; 4.2691x over previous
import jax
import jax.numpy as jnp
from jax.experimental import pallas as pl
from jax.experimental.pallas import tpu as pltpu

NUM_EXPERTS = 8
TOP_K = 2
SEQ = 2048
D_MODEL = 1024
MLP_DIM = 4096

BT = 256
FB = 1024
NF = MLP_DIM // FB
NB = (SEQ * TOP_K) // BT + NUM_EXPERTS
R = NB * BT


def _moe_mlp_kernel(s_ref, x_ref, w0_ref, w1_ref, wo_ref, o_ref, acc_ref):
    j = pl.program_id(0)
    i = pl.program_id(1)
    nb = s_ref[NB]

    @pl.when(i < nb)
    def _():
        x = x_ref[...]
        h0 = jnp.dot(x, w0_ref[0], preferred_element_type=jnp.float32)
        h1 = jnp.dot(x, w1_ref[0], preferred_element_type=jnp.float32)
        h = jax.nn.silu(h0) * h1
        y = jnp.dot(h, wo_ref[0], preferred_element_type=jnp.float32)

        @pl.when(j == 0)
        def _():
            acc_ref[pl.ds(i * BT, BT), :] = y

        @pl.when(j > 0)
        def _():
            acc_ref[pl.ds(i * BT, BT), :] += y

        @pl.when(j == NF - 1)
        def _():
            o_ref[...] = acc_ref[pl.ds(i * BT, BT), :].astype(jnp.bfloat16)


def _grouped_mlp(meta, x_g, wi_0, wi_1, wo):
    grid_spec = pltpu.PrefetchScalarGridSpec(
        num_scalar_prefetch=1,
        grid=(NF, NB),
        in_specs=[
            pl.BlockSpec((BT, D_MODEL), lambda j, i, s: (i, 0)),
            pl.BlockSpec((1, D_MODEL, FB), lambda j, i, s: (s[i], 0, j)),
            pl.BlockSpec((1, D_MODEL, FB), lambda j, i, s: (s[i], 0, j)),
            pl.BlockSpec((1, FB, D_MODEL), lambda j, i, s: (s[i], j, 0)),
        ],
        out_specs=pl.BlockSpec(
            (BT, D_MODEL),
            lambda j, i, s: (jnp.where(j == NF - 1, i, 0), 0)),
        scratch_shapes=[pltpu.VMEM((R, D_MODEL), jnp.float32)],
    )
    return pl.pallas_call(
        _moe_mlp_kernel,
        grid_spec=grid_spec,
        out_shape=jax.ShapeDtypeStruct((R, D_MODEL), jnp.bfloat16),
        compiler_params=pltpu.CompilerParams(
            dimension_semantics=("arbitrary", "arbitrary"),
        ),
    )(meta, x_g, wi_0, wi_1, wo)


def kernel(inputs, gate_w, wi_0, wi_1, wo):
    x = inputs.reshape(SEQ, D_MODEL)

    logits = x @ gate_w
    e0 = jnp.argmax(logits, axis=-1).astype(jnp.int32)
    v0 = jnp.max(logits, axis=-1)
    masked = jnp.where(
        jax.nn.one_hot(e0, NUM_EXPERTS, dtype=jnp.bool_), -jnp.inf, logits)
    e1 = jnp.argmax(masked, axis=-1).astype(jnp.int32)
    v1 = jnp.max(masked, axis=-1)
    p1 = jax.nn.sigmoid(v1 - v0)
    top_w = jnp.stack([1.0 - p1, p1], axis=-1)
    experts_flat = jnp.stack([e0, e1], axis=-1).reshape(-1)

    onehot = (experts_flat[:, None] ==
              jnp.arange(NUM_EXPERTS)[None, :]).astype(jnp.int32)
    csum = jnp.cumsum(onehot, axis=0)
    counts = csum[-1]
    ranks = jnp.take_along_axis(csum, experts_flat[:, None], axis=1)[:, 0] - 1
    padded_counts = ((counts + BT - 1) // BT) * BT
    padded_offsets = jnp.concatenate(
        [jnp.zeros((1,), jnp.int32), jnp.cumsum(padded_counts)[:-1]]
    ).astype(jnp.int32)
    pos = padded_offsets[experts_flat] + ranks
    num_blocks = (padded_offsets[-1] + padded_counts[-1]) // BT

    token_of = jnp.arange(SEQ * TOP_K, dtype=jnp.int32) // TOP_K
    gather_idx = jnp.zeros((R,), jnp.int32).at[pos].set(
        token_of, unique_indices=True, mode="promise_in_bounds")
    block_expert = (
        jnp.searchsorted(padded_offsets,
                         jnp.arange(NB, dtype=jnp.int32) * BT, side="right")
        - 1
    ).astype(jnp.int32)
    meta = jnp.concatenate(
        [block_expert, num_blocks.reshape(1).astype(jnp.int32)])

    x_g = x.astype(jnp.bfloat16)[gather_idx]

    y_g = _grouped_mlp(meta, x_g, wi_0, wi_1, wo)

    out = (top_w[:, :, None] * y_g[pos.reshape(SEQ, TOP_K)]).sum(axis=1)
    return out.reshape(1, SEQ, D_MODEL)


# speedup vs baseline: 4.6121x; 1.0803x over previous
import jax
import jax.numpy as jnp
from jax.experimental import pallas as pl
from jax.experimental.pallas import tpu as pltpu

NUM_EXPERTS = 8
TOP_K = 2
SEQ = 2048
D_MODEL = 1024
MLP_DIM = 4096

BT = 256
FB = 1024
NF = MLP_DIM // FB
NB = (SEQ * TOP_K) // BT + NUM_EXPERTS
R = NB * BT
NA = SEQ * TOP_K
NAB = NA // BT


def _moe_mlp_kernel(s_ref, x_ref, w0_ref, w1_ref, wo_ref, o_ref, acc_ref):
    j = pl.program_id(0)
    i = pl.program_id(1)
    nb = s_ref[NB]

    @pl.when(jnp.logical_and(j < NF, i < nb))
    def _():
        x = x_ref[...]
        h0 = jnp.dot(x, w0_ref[0], preferred_element_type=jnp.float32)
        h1 = jnp.dot(x, w1_ref[0], preferred_element_type=jnp.float32)
        h = jax.nn.silu(h0) * h1
        y = jnp.dot(h, wo_ref[0], preferred_element_type=jnp.float32)

        @pl.when(j == 0)
        def _():
            acc_ref[pl.ds(i * BT, BT), :] = y

        @pl.when(j > 0)
        def _():
            acc_ref[pl.ds(i * BT, BT), :] += y

    @pl.when(jnp.logical_and(j == NF, i < NAB))
    def _():
        base = i * BT
        for rb in range(BT // 8):
            rows = [
                acc_ref[pl.ds(s_ref[NB + 1 + base + rb * 8 + u], 1), :]
                for u in range(8)
            ]
            o_ref[rb * 8:(rb + 1) * 8, :] = jnp.concatenate(
                rows, axis=0).astype(jnp.bfloat16)


def _grouped_mlp(meta, x_g, wi_0, wi_1, wo):
    grid_spec = pltpu.PrefetchScalarGridSpec(
        num_scalar_prefetch=1,
        grid=(NF + 1, NB),
        in_specs=[
            pl.BlockSpec((BT, D_MODEL),
                         lambda j, i, s: (jnp.where(j == NF, 0, i), 0)),
            pl.BlockSpec((1, D_MODEL, FB),
                         lambda j, i, s: (jnp.where(j == NF,
                                                    NUM_EXPERTS - 1, s[i]),
                                          0, jnp.minimum(j, NF - 1))),
            pl.BlockSpec((1, D_MODEL, FB),
                         lambda j, i, s: (jnp.where(j == NF,
                                                    NUM_EXPERTS - 1, s[i]),
                                          0, jnp.minimum(j, NF - 1))),
            pl.BlockSpec((1, FB, D_MODEL),
                         lambda j, i, s: (jnp.where(j == NF,
                                                    NUM_EXPERTS - 1, s[i]),
                                          jnp.minimum(j, NF - 1), 0)),
        ],
        out_specs=pl.BlockSpec(
            (BT, D_MODEL),
            lambda j, i, s: (jnp.where(j == NF,
                                       jnp.minimum(i, NAB - 1), 0), 0)),
        scratch_shapes=[pltpu.VMEM((R, D_MODEL), jnp.float32)],
    )
    return pl.pallas_call(
        _moe_mlp_kernel,
        grid_spec=grid_spec,
        out_shape=jax.ShapeDtypeStruct((NA, D_MODEL), jnp.bfloat16),
        compiler_params=pltpu.CompilerParams(
            dimension_semantics=("arbitrary", "arbitrary"),
        ),
    )(meta, x_g, wi_0, wi_1, wo)


def kernel(inputs, gate_w, wi_0, wi_1, wo):
    x = inputs.reshape(SEQ, D_MODEL)

    logits = x @ gate_w
    e0 = jnp.argmax(logits, axis=-1).astype(jnp.int32)
    v0 = jnp.max(logits, axis=-1)
    masked = jnp.where(
        jax.nn.one_hot(e0, NUM_EXPERTS, dtype=jnp.bool_), -jnp.inf, logits)
    e1 = jnp.argmax(masked, axis=-1).astype(jnp.int32)
    v1 = jnp.max(masked, axis=-1)
    p1 = jax.nn.sigmoid(v1 - v0)
    top_w = jnp.stack([1.0 - p1, p1], axis=-1)
    experts_flat = jnp.stack([e0, e1], axis=-1).reshape(-1)

    onehot = (experts_flat[:, None] ==
              jnp.arange(NUM_EXPERTS)[None, :]).astype(jnp.int32)
    csum = jnp.cumsum(onehot, axis=0)
    counts = csum[-1]
    ranks = jnp.take_along_axis(csum, experts_flat[:, None], axis=1)[:, 0] - 1
    padded_counts = ((counts + BT - 1) // BT) * BT
    padded_offsets = jnp.concatenate(
        [jnp.zeros((1,), jnp.int32), jnp.cumsum(padded_counts)[:-1]]
    ).astype(jnp.int32)
    pos = padded_offsets[experts_flat] + ranks
    num_blocks = (padded_offsets[-1] + padded_counts[-1]) // BT

    token_of = jnp.arange(NA, dtype=jnp.int32) // TOP_K
    gather_idx = jnp.zeros((R,), jnp.int32).at[pos].set(
        token_of, unique_indices=True, mode="promise_in_bounds")
    block_expert = (
        jnp.searchsorted(padded_offsets,
                         jnp.arange(NB, dtype=jnp.int32) * BT, side="right")
        - 1
    ).astype(jnp.int32)
    meta = jnp.concatenate(
        [block_expert, num_blocks.reshape(1).astype(jnp.int32), pos])

    x_g = x.astype(jnp.bfloat16)[gather_idx]

    y_pairs = _grouped_mlp(meta, x_g, wi_0, wi_1, wo)

    out = (top_w[:, :, None] *
           y_pairs.reshape(SEQ, TOP_K, D_MODEL)).sum(axis=1)
    return out.reshape(1, SEQ, D_MODEL)


# speedup vs baseline: 5.1665x; 1.1202x over previous
import jax
import jax.numpy as jnp
from jax.experimental import pallas as pl
from jax.experimental.pallas import tpu as pltpu

NUM_EXPERTS = 8
TOP_K = 2
SEQ = 2048
D_MODEL = 1024
MLP_DIM = 4096

BT = 256
FB = 1024
NF = MLP_DIM // FB
NB = (SEQ * TOP_K) // BT + NUM_EXPERTS
R = NB * BT
NA = SEQ * TOP_K
NTB = SEQ // BT


def _moe_mlp_kernel(s_ref, tw_ref, x_ref, w0_ref, w1_ref, wo_ref, o_ref,
                    acc_ref):
    j = pl.program_id(0)
    i = pl.program_id(1)
    nb = s_ref[NB]

    @pl.when(jnp.logical_and(j < NF, i < nb))
    def _():
        x = x_ref[...]
        h0 = jnp.dot(x, w0_ref[0], preferred_element_type=jnp.float32)
        h1 = jnp.dot(x, w1_ref[0], preferred_element_type=jnp.float32)
        h = jax.nn.silu(h0) * h1
        y = jnp.dot(h, wo_ref[0], preferred_element_type=jnp.float32)

        @pl.when(j == 0)
        def _():
            acc_ref[pl.ds(i * BT, BT), :] = y

        @pl.when(j > 0)
        def _():
            acc_ref[pl.ds(i * BT, BT), :] += y

    @pl.when(jnp.logical_and(j == NF, i < NTB))
    def _():
        base = i * BT
        for rb in range(BT // 8):
            rows = []
            for u in range(8):
                t = base + rb * 8 + u
                r0 = acc_ref[pl.ds(s_ref[NB + 1 + 2 * t], 1), :]
                r1 = acc_ref[pl.ds(s_ref[NB + 1 + 2 * t + 1], 1), :]
                rows.append(tw_ref[2 * t] * r0 + tw_ref[2 * t + 1] * r1)
            o_ref[rb * 8:(rb + 1) * 8, :] = jnp.concatenate(rows, axis=0)


def _grouped_mlp(meta, tw, x_g, wi_0, wi_1, wo):
    grid_spec = pltpu.PrefetchScalarGridSpec(
        num_scalar_prefetch=2,
        grid=(NF + 1, NB),
        in_specs=[
            pl.BlockSpec((BT, D_MODEL),
                         lambda j, i, s, w: (jnp.where(j == NF, 0, i), 0)),
            pl.BlockSpec((1, D_MODEL, FB),
                         lambda j, i, s, w: (jnp.where(j == NF,
                                                       NUM_EXPERTS - 1, s[i]),
                                             0, jnp.minimum(j, NF - 1))),
            pl.BlockSpec((1, D_MODEL, FB),
                         lambda j, i, s, w: (jnp.where(j == NF,
                                                       NUM_EXPERTS - 1, s[i]),
                                             0, jnp.minimum(j, NF - 1))),
            pl.BlockSpec((1, FB, D_MODEL),
                         lambda j, i, s, w: (jnp.where(j == NF,
                                                       NUM_EXPERTS - 1, s[i]),
                                             jnp.minimum(j, NF - 1), 0)),
        ],
        out_specs=pl.BlockSpec(
            (BT, D_MODEL),
            lambda j, i, s, w: (jnp.where(j == NF,
                                          jnp.minimum(i, NTB - 1), 0), 0)),
        scratch_shapes=[pltpu.VMEM((R, D_MODEL), jnp.float32)],
    )
    return pl.pallas_call(
        _moe_mlp_kernel,
        grid_spec=grid_spec,
        out_shape=jax.ShapeDtypeStruct((SEQ, D_MODEL), jnp.float32),
        compiler_params=pltpu.CompilerParams(
            dimension_semantics=("arbitrary", "arbitrary"),
        ),
    )(meta, tw, x_g, wi_0, wi_1, wo)


def kernel(inputs, gate_w, wi_0, wi_1, wo):
    x = inputs.reshape(SEQ, D_MODEL)

    logits = x @ gate_w
    e0 = jnp.argmax(logits, axis=-1).astype(jnp.int32)
    v0 = jnp.max(logits, axis=-1)
    masked = jnp.where(
        jax.nn.one_hot(e0, NUM_EXPERTS, dtype=jnp.bool_), -jnp.inf, logits)
    e1 = jnp.argmax(masked, axis=-1).astype(jnp.int32)
    v1 = jnp.max(masked, axis=-1)
    p1 = jax.nn.sigmoid(v1 - v0)
    top_w = jnp.stack([1.0 - p1, p1], axis=-1)
    experts_flat = jnp.stack([e0, e1], axis=-1).reshape(-1)

    onehot = (experts_flat[:, None] ==
              jnp.arange(NUM_EXPERTS)[None, :]).astype(jnp.int32)
    csum = jnp.cumsum(onehot, axis=0)
    counts = csum[-1]
    ranks = jnp.take_along_axis(csum, experts_flat[:, None], axis=1)[:, 0] - 1
    padded_counts = ((counts + BT - 1) // BT) * BT
    padded_offsets = jnp.concatenate(
        [jnp.zeros((1,), jnp.int32), jnp.cumsum(padded_counts)[:-1]]
    ).astype(jnp.int32)
    pos = padded_offsets[experts_flat] + ranks
    num_blocks = (padded_offsets[-1] + padded_counts[-1]) // BT

    token_of = jnp.arange(NA, dtype=jnp.int32) // TOP_K
    gather_idx = jnp.zeros((R,), jnp.int32).at[pos].set(
        token_of, unique_indices=True, mode="promise_in_bounds")
    block_expert = (
        jnp.searchsorted(padded_offsets,
                         jnp.arange(NB, dtype=jnp.int32) * BT, side="right")
        - 1
    ).astype(jnp.int32)
    meta = jnp.concatenate(
        [block_expert, num_blocks.reshape(1).astype(jnp.int32), pos])

    x_g = x.astype(jnp.bfloat16)[gather_idx]

    out = _grouped_mlp(meta, top_w.reshape(-1), x_g, wi_0, wi_1, wo)
    return out.reshape(1, SEQ, D_MODEL)


# speedup vs baseline: 5.3055x; 1.0269x over previous
import jax
import jax.numpy as jnp
from jax.experimental import pallas as pl
from jax.experimental.pallas import tpu as pltpu

NUM_EXPERTS = 8
TOP_K = 2
SEQ = 2048
D_MODEL = 1024
MLP_DIM = 4096

BT = 256
FB = 1024
NF = MLP_DIM // FB
NB = (SEQ * TOP_K) // BT + NUM_EXPERTS
R = NB * BT
NA = SEQ * TOP_K
NTB = SEQ // BT


def _moe_mlp_kernel(s_ref, tw_ref, x_ref, w0_ref, w1_ref, wo_ref, o_ref,
                    acc_ref):
    j = pl.program_id(0)
    i = pl.program_id(1)
    nb = s_ref[NB]

    @pl.when(jnp.logical_and(j < NF, i < nb))
    def _():
        x = x_ref[pl.ds(i * BT, BT), :]
        h0 = jnp.dot(x, w0_ref[0], preferred_element_type=jnp.float32)
        h1 = jnp.dot(x, w1_ref[0], preferred_element_type=jnp.float32)
        h = jax.nn.silu(h0) * h1
        y = jnp.dot(h, wo_ref[0], preferred_element_type=jnp.float32)

        @pl.when(j == 0)
        def _():
            acc_ref[pl.ds(i * BT, BT), :] = y

        @pl.when(j > 0)
        def _():
            acc_ref[pl.ds(i * BT, BT), :] += y

    @pl.when(jnp.logical_and(j == NF, i < NTB))
    def _():
        base = i * BT
        for rb in range(BT // 8):
            rows = []
            for u in range(8):
                t = base + rb * 8 + u
                r0 = acc_ref[pl.ds(s_ref[NB + 1 + 2 * t], 1), :]
                r1 = acc_ref[pl.ds(s_ref[NB + 1 + 2 * t + 1], 1), :]
                rows.append(tw_ref[2 * t] * r0 + tw_ref[2 * t + 1] * r1)
            o_ref[rb * 8:(rb + 1) * 8, :] = jnp.concatenate(rows, axis=0)


def _grouped_mlp(meta, tw, x_g, wi_0, wi_1, wo):
    grid_spec = pltpu.PrefetchScalarGridSpec(
        num_scalar_prefetch=2,
        grid=(NF + 1, NB),
        in_specs=[
            pl.BlockSpec((R, D_MODEL), lambda j, i, s, w: (0, 0)),
            pl.BlockSpec((1, D_MODEL, FB),
                         lambda j, i, s, w: (jnp.where(j == NF,
                                                    NUM_EXPERTS - 1, s[i]),
                                          0, jnp.minimum(j, NF - 1))),
            pl.BlockSpec((1, D_MODEL, FB),
                         lambda j, i, s, w: (jnp.where(j == NF,
                                                    NUM_EXPERTS - 1, s[i]),
                                          0, jnp.minimum(j, NF - 1))),
            pl.BlockSpec((1, FB, D_MODEL),
                         lambda j, i, s, w: (jnp.where(j == NF,
                                                    NUM_EXPERTS - 1, s[i]),
                                          jnp.minimum(j, NF - 1), 0)),
        ],
        out_specs=pl.BlockSpec(
            (BT, D_MODEL),
            lambda j, i, s, w: (jnp.where(j == NF,
                                       jnp.minimum(i, NTB - 1), 0), 0)),
        scratch_shapes=[pltpu.VMEM((R, D_MODEL), jnp.float32)],
    )
    return pl.pallas_call(
        _moe_mlp_kernel,
        grid_spec=grid_spec,
        out_shape=jax.ShapeDtypeStruct((SEQ, D_MODEL), jnp.float32),
        compiler_params=pltpu.CompilerParams(
            dimension_semantics=("arbitrary", "arbitrary"),
            vmem_limit_bytes=67000000,
        ),
    )(meta, tw, x_g, wi_0, wi_1, wo)


def kernel(inputs, gate_w, wi_0, wi_1, wo):
    x = inputs.reshape(SEQ, D_MODEL)

    logits = x @ gate_w
    e0 = jnp.argmax(logits, axis=-1).astype(jnp.int32)
    v0 = jnp.max(logits, axis=-1)
    masked = jnp.where(
        jax.nn.one_hot(e0, NUM_EXPERTS, dtype=jnp.bool_), -jnp.inf, logits)
    e1 = jnp.argmax(masked, axis=-1).astype(jnp.int32)
    v1 = jnp.max(masked, axis=-1)
    p1 = jax.nn.sigmoid(v1 - v0)
    top_w = jnp.stack([1.0 - p1, p1], axis=-1)
    experts_flat = jnp.stack([e0, e1], axis=-1).reshape(-1)

    onehot = (experts_flat[:, None] ==
              jnp.arange(NUM_EXPERTS)[None, :]).astype(jnp.int32)
    csum = jnp.cumsum(onehot, axis=0)
    counts = csum[-1]
    ranks = jnp.take_along_axis(csum, experts_flat[:, None], axis=1)[:, 0] - 1
    padded_counts = ((counts + BT - 1) // BT) * BT
    padded_offsets = jnp.concatenate(
        [jnp.zeros((1,), jnp.int32), jnp.cumsum(padded_counts)[:-1]]
    ).astype(jnp.int32)
    pos = padded_offsets[experts_flat] + ranks
    num_blocks = (padded_offsets[-1] + padded_counts[-1]) // BT

    token_of = jnp.arange(NA, dtype=jnp.int32) // TOP_K
    gather_idx = jnp.zeros((R,), jnp.int32).at[pos].set(
        token_of, unique_indices=True, mode="promise_in_bounds")
    block_expert = (
        jnp.searchsorted(padded_offsets,
                         jnp.arange(NB, dtype=jnp.int32) * BT, side="right")
        - 1
    ).astype(jnp.int32)
    meta = jnp.concatenate(
        [block_expert, num_blocks.reshape(1).astype(jnp.int32), pos])

    x_g = x.astype(jnp.bfloat16)[gather_idx]

    out = _grouped_mlp(meta, top_w.reshape(-1), x_g, wi_0, wi_1, wo)
    return out.reshape(1, SEQ, D_MODEL)
